# baseline (speedup 1.0000x reference)
import functools

import jax
import jax.numpy as jnp
from jax import lax
from jax.experimental import pallas as pl
from jax.experimental.pallas import tpu as pltpu
from jax.experimental.pallas import tpu_sc as plsc

_NE = 8192
_D = 64
_M = 16384
_BM = 1024
_DP = 128


_NI = 2
_NC = 4096


def _argmin_body(x_ref, w_ref, wsq_ref, idx_ref):
    x = jnp.concatenate([x_ref[i] for i in range(_NI)], axis=1)
    x2 = x * -2.0
    xsq = jnp.sum(x * x, axis=0, keepdims=True)
    best_val = None
    for c in range(_NE // _NC):
        wc = w_ref[pl.ds(c * _NC, _NC), :]
        mm2 = jax.lax.dot_general(
            wc, x2, (((1,), (0,)), ((), ())),
            preferred_element_type=jnp.float32,
        )
        d = (wsq_ref[pl.ds(c * _NC, _NC), :] + xsq) + mm2
        val = jnp.min(d, axis=0)
        loc = jnp.argmin(d, axis=0).astype(jnp.int32) + jnp.int32(c * _NC)
        if best_val is None:
            best_val, best_idx = val, loc
        else:
            upd = val < best_val
            best_val = jnp.where(upd, val, best_val)
            best_idx = jnp.where(upd, loc, best_idx)
    idx_ref[...] = best_idx


def _argmin_call(x3, w, wsq):
    bm = _NI * 1024
    return pl.pallas_call(
        _argmin_body,
        grid=(_M // bm,),
        in_specs=[
            pl.BlockSpec((_NI, _D, 1024), lambda i: (i, 0, 0)),
            pl.BlockSpec((_NE, _D), lambda i: (0, 0)),
            pl.BlockSpec((_NE, 1), lambda i: (0, 0)),
        ],
        out_specs=pl.BlockSpec((bm,), lambda i: (i,)),
        out_shape=jax.ShapeDtypeStruct((_M,), jnp.int32),
    )(x3, w, wsq)


def _gather_call(table, idx):
    mesh = plsc.VectorSubcoreMesh(core_axis_name="c", subcore_axis_name="s")
    nw = mesh.num_cores * mesh.num_subcores
    bw = _M // nw
    nch = bw // _DP
    idx3 = idx.reshape(nw, nch, _DP)

    @functools.partial(
        pl.kernel,
        out_type=jax.ShapeDtypeStruct((_M, _D), jnp.float32),
        mesh=mesh,
        scratch_types=[
            pltpu.VMEM((nch, _DP), jnp.int32),
            pltpu.VMEM((bw, _D), jnp.float32),
            pltpu.SemaphoreType.DMA,
        ],
        compiler_params=pltpu.CompilerParams(use_tc_tiling_on_sc=False),
    )
    def gk(table_hbm, idx_hbm, out_hbm, idx_v, rows_v, sem):
        wid = lax.axis_index("s") * mesh.num_cores + lax.axis_index("c")
        pltpu.sync_copy(idx_hbm.at[wid], idx_v)
        cps = [
            pltpu.async_copy(
                table_hbm.at[idx_v.at[j]],
                rows_v.at[pl.ds(j * _DP, _DP)],
                sem,
            )
            for j in range(nch)
        ]
        for c in cps:
            c.wait()
        pltpu.sync_copy(rows_v, out_hbm.at[pl.ds(wid * bw, bw)])

    return gk(table, idx3)


def kernel(inputs, weight):
    x3 = inputs.reshape(16, _D, 32 * 32)
    wsq = jnp.sum(weight ** 2, axis=1)[:, None]
    idx = _argmin_call(x3, weight, wsq)
    q = _gather_call(weight, idx)
    return q.reshape(16, 32, 32, _D).transpose(0, 3, 1, 2)


# speedup vs baseline: 1.0979x; 1.0979x over previous
import functools

import jax
import jax.numpy as jnp
from jax import lax
from jax.experimental import pallas as pl
from jax.experimental.pallas import tpu as pltpu
from jax.experimental.pallas import tpu_sc as plsc

_NE = 8192
_D = 64
_M = 16384
_BM = 1024
_DP = 128


def _argmin_body(x_ref, w_ref, idx_ref, wsq_s):
    @pl.when(pl.program_id(0) == 0)
    def _():
        w = w_ref[...]
        wsq_s[...] = jnp.sum(w * w, axis=1, keepdims=True)

    x = x_ref[0]
    x2 = x * -2.0
    xsq = jnp.sum(x * x, axis=0, keepdims=True)
    mm2 = jax.lax.dot_general(
        w_ref[...], x2, (((1,), (0,)), ((), ())),
        preferred_element_type=jnp.float32,
    )
    d = (wsq_s[...] + xsq) + mm2
    idx_ref[...] = jnp.argmin(d, axis=0).astype(jnp.int32)


def _argmin_call(x3, w):
    nimg, _, npix = x3.shape
    per = npix // _BM
    return pl.pallas_call(
        _argmin_body,
        grid=(_M // _BM,),
        in_specs=[
            pl.BlockSpec((1, _D, _BM), lambda i: (i // per, 0, i % per)),
            pl.BlockSpec((_NE, _D), lambda i: (0, 0)),
        ],
        out_specs=pl.BlockSpec((_BM,), lambda i: (i,)),
        out_shape=jax.ShapeDtypeStruct((_M,), jnp.int32),
        scratch_shapes=[pltpu.VMEM((_NE, 1), jnp.float32)],
    )(x3, w)


def _gather_call(table, idx):
    mesh = plsc.VectorSubcoreMesh(core_axis_name="c", subcore_axis_name="s")
    nw = mesh.num_cores * mesh.num_subcores
    bw = _M // nw
    nch = bw // _DP
    idx3 = idx.reshape(nw, nch, _DP)

    @functools.partial(
        pl.kernel,
        out_type=jax.ShapeDtypeStruct((_M, _D), jnp.float32),
        mesh=mesh,
        scratch_types=[
            pltpu.VMEM((nch, _DP), jnp.int32),
            pltpu.VMEM((bw, _D), jnp.float32),
            pltpu.SemaphoreType.DMA,
        ],
        compiler_params=pltpu.CompilerParams(use_tc_tiling_on_sc=False),
    )
    def gk(table_hbm, idx_hbm, out_hbm, idx_v, rows_v, sem):
        wid = lax.axis_index("s") * mesh.num_cores + lax.axis_index("c")
        pltpu.sync_copy(idx_hbm.at[wid], idx_v)
        cps = [
            pltpu.async_copy(
                table_hbm.at[idx_v.at[j]],
                rows_v.at[pl.ds(j * _DP, _DP)],
                sem,
            )
            for j in range(nch)
        ]
        for c in cps:
            c.wait()
        pltpu.sync_copy(rows_v, out_hbm.at[pl.ds(wid * bw, bw)])

    return gk(table, idx3)


def kernel(inputs, weight):
    x3 = inputs.reshape(16, _D, 32 * 32)
    idx = _argmin_call(x3, weight)
    q = _gather_call(weight, idx)
    return q.reshape(16, 32, 32, _D).transpose(0, 3, 1, 2)
